# Initial kernel scaffold; baseline (speedup 1.0000x reference)
import jax
import jax.numpy as jnp
from jax.experimental import pallas as pl


def kernel(x, edge_index, W1, b1, W2, b2, Wm1, bm1, Wm2, bm2):
    raise NotImplementedError("write your pallas kernel here")


# speedup vs baseline: 4.1324x; 4.1324x over previous
import functools

import jax
import jax.numpy as jnp
from jax import lax
from jax.experimental import pallas as pl
from jax.experimental.pallas import tpu as pltpu
from jax.experimental.pallas import tpu_sc as plsc

N = 10000
E = 320000
IN = 128
HID = 128
EMB = 64

NC = 2
NS = 16
L = 16
NW = NC * NS
EPW = E // NW
N_PAD = 10240
RPS = N_PAD // NS
CE = 80

_mesh = plsc.VectorSubcoreMesh(core_axis_name="c", subcore_axis_name="s")


CD = 2000


def _deg_body(dst_hbm, zero_hbm, out_hbm, dchunk, hist_v):
    c = lax.axis_index("c")
    s = lax.axis_index("s")
    wid = s * NC + c
    pltpu.sync_copy(zero_hbm, hist_v)
    ones = jnp.ones((L,), jnp.float32)
    base = wid * EPW

    def chunk(ci, carry):
        pltpu.sync_copy(dst_hbm.at[pl.ds(base + ci * CD, CD)], dchunk)

        def grp(g, carry2):
            idx = dchunk[pl.ds(g * L, L)]
            plsc.addupdate_scatter(hist_v, [idx], ones)
            return carry2

        return lax.fori_loop(0, CD // L, grp, carry)

    lax.fori_loop(0, EPW // CD, chunk, 0)
    pltpu.sync_copy(hist_v, out_hbm.at[pl.ds(wid * N_PAD, N_PAD)])


_deg_call = pl.kernel(
    _deg_body,
    out_type=jax.ShapeDtypeStruct((NW * N_PAD,), jnp.float32),
    mesh=_mesh,
    compiler_params=pltpu.CompilerParams(needs_layout_passes=False),
    scratch_types=[
        pltpu.VMEM((CD,), jnp.int32),
        pltpu.VMEM((N_PAD,), jnp.float32),
    ],
)


def _conv_body(u_hbm, src_hbm, dst_hbm, zero_hbm, out_hbm,
               srcv, dstv, rows, sem, acc):
    c = lax.axis_index("c")
    s = lax.axis_index("s")
    wid = s * NC + c
    pltpu.sync_copy(zero_hbm, acc.at[pl.ds(s * RPS, RPS)])
    plsc.subcore_barrier()

    def chunk(ci, carry):
        base = wid * EPW + ci * CE
        pltpu.sync_copy(src_hbm.at[pl.ds(base, CE)], srcv)
        pltpu.sync_copy(dst_hbm.at[pl.ds(base, CE)], dstv)
        pltpu.async_copy(u_hbm.at[srcv], rows, sem).wait()
        pltpu.sync_copy(rows, acc.at[dstv], add=True)
        return carry

    lax.fori_loop(0, EPW // CE, chunk, 0)
    plsc.subcore_barrier()
    pltpu.sync_copy(acc.at[pl.ds(s * RPS, RPS)],
                    out_hbm.at[c, pl.ds(s * RPS, RPS)])


def _make_conv_call(d):
    return pl.kernel(
        _conv_body,
        out_type=jax.ShapeDtypeStruct((NC, N_PAD, d), jnp.float32),
        mesh=_mesh,
        scratch_types=[
            pltpu.VMEM((CE,), jnp.int32),
            pltpu.VMEM((CE,), jnp.int32),
            pltpu.VMEM((CE, d), jnp.float32),
            pltpu.SemaphoreType.DMA,
            pltpu.MemorySpace.VMEM_SHARED((N_PAD, d), jnp.float32),
        ],
    )


_conv_call_hid = _make_conv_call(HID)


def _mlp_body(zs_hbm, zd_hbm, src_hbm, dst_hbm, w2_hbm, bm2_hbm, out_hbm,
              srcv, dstv, abuf, bbuf, w2v, bm2v, outv, sem):
    c = lax.axis_index("c")
    s = lax.axis_index("s")
    wid = s * NC + c
    pltpu.sync_copy(w2_hbm, w2v)
    pltpu.sync_copy(bm2_hbm, bm2v)
    ng = CE // L

    def chunk(ci, carry):
        base = wid * EPW + ci * CE
        pltpu.sync_copy(src_hbm.at[pl.ds(base, CE)], srcv)
        pltpu.sync_copy(dst_hbm.at[pl.ds(base, CE)], dstv)
        pltpu.async_copy(zs_hbm.at[srcv], abuf, sem).wait()
        pltpu.async_copy(zd_hbm.at[dstv], bbuf, sem).wait()

        def kbody(k, accs):
            kb = jnp.full((L,), k, jnp.int32)
            w2k = plsc.load_gather(w2v, [kb])
            out = []
            for g in range(ng):
                eids = lax.iota(jnp.int32, L) + g * L
                a = plsc.load_gather(abuf, [eids, kb])
                b = plsc.load_gather(bbuf, [eids, kb])
                out.append(accs[g] + jnp.maximum(a + b, 0.0) * w2k)
            return tuple(out)

        accs = lax.fori_loop(
            0, HID, kbody,
            tuple(jnp.zeros((L,), jnp.float32) for _ in range(ng)))
        bm2 = bm2v[...]
        for g in range(ng):
            outv[pl.ds(g * L, L)] = accs[g] + bm2
        pltpu.sync_copy(outv, out_hbm.at[pl.ds(base, CE)])
        return carry

    lax.fori_loop(0, EPW // CE, chunk, 0)


_mlp_call = pl.kernel(
    _mlp_body,
    out_type=jax.ShapeDtypeStruct((E,), jnp.float32),
    mesh=_mesh,
    compiler_params=pltpu.CompilerParams(needs_layout_passes=False),
    scratch_types=[
        pltpu.VMEM((CE,), jnp.int32),
        pltpu.VMEM((CE,), jnp.int32),
        pltpu.VMEM((CE, HID), jnp.float32),
        pltpu.VMEM((CE, HID), jnp.float32),
        pltpu.VMEM((HID,), jnp.float32),
        pltpu.VMEM((L,), jnp.float32),
        pltpu.VMEM((CE,), jnp.float32),
        pltpu.SemaphoreType.DMA,
    ],
)


_B = 1000


def _tca_body(hist_ref, x_ref, w1_ref, u0_ref, dinv_ref):
    deg = jnp.sum(hist_ref[...], axis=0) + 1.0
    dinv = lax.rsqrt(deg)
    h0 = jnp.dot(x_ref[...], w1_ref[...], preferred_element_type=jnp.float32)
    u0_ref[...] = h0 * dinv
    dinv_ref[...] = dinv


def _tca_call(hist, x, w1):
    return pl.pallas_call(
        _tca_body,
        grid=(N // _B,),
        in_specs=[
            pl.BlockSpec((NW, _B, 1), lambda i: (0, i, 0)),
            pl.BlockSpec((_B, IN), lambda i: (i, 0)),
            pl.BlockSpec((IN, HID), lambda i: (0, 0)),
        ],
        out_specs=[
            pl.BlockSpec((_B, HID), lambda i: (i, 0)),
            pl.BlockSpec((_B, 1), lambda i: (i, 0)),
        ],
        out_shape=[
            jax.ShapeDtypeStruct((N, HID), jnp.float32),
            jax.ShapeDtypeStruct((N, 1), jnp.float32),
        ],
    )(hist, x, w1)


def _tcb_body(acc_ref, u0_ref, dinv_ref, b1_ref, w2_ref, u1_ref):
    dinv = dinv_ref[...]
    pre = (acc_ref[0] + acc_ref[1] + u0_ref[...]) * dinv + b1_ref[...]
    h = jnp.maximum(pre, 0.0)
    u1 = jnp.dot(h, w2_ref[...], preferred_element_type=jnp.float32) * dinv
    u1_ref[...] = jnp.concatenate([u1, jnp.zeros_like(u1)], axis=1)


def _tcb_call(acc, u0, dinv, b1, w2):
    return pl.pallas_call(
        _tcb_body,
        grid=(N // _B,),
        in_specs=[
            pl.BlockSpec((NC, _B, HID), lambda i: (0, i, 0)),
            pl.BlockSpec((_B, HID), lambda i: (i, 0)),
            pl.BlockSpec((_B, 1), lambda i: (i, 0)),
            pl.BlockSpec((1, HID), lambda i: (0, 0)),
            pl.BlockSpec((HID, EMB), lambda i: (0, 0)),
        ],
        out_specs=pl.BlockSpec((_B, 2 * EMB), lambda i: (i, 0)),
        out_shape=jax.ShapeDtypeStruct((N, 2 * EMB), jnp.float32),
    )(acc, u0, dinv, b1, w2)


def _tcc_body(acc_ref, u1_ref, dinv_ref, b2_ref, wm1_ref, bm1_ref,
              zs_ref, zd_ref):
    zw = acc_ref[0] + acc_ref[1] + u1_ref[...]
    z = zw[:, :EMB] * dinv_ref[...] + b2_ref[...]
    wm1 = wm1_ref[...]
    zs_ref[...] = jnp.dot(z, wm1[:EMB], preferred_element_type=jnp.float32)
    zd_ref[...] = (jnp.dot(z, wm1[EMB:], preferred_element_type=jnp.float32)
                   + bm1_ref[...])


def _tcc_call(acc, u1, dinv, b2, wm1, bm1):
    return pl.pallas_call(
        _tcc_body,
        grid=(N // _B,),
        in_specs=[
            pl.BlockSpec((NC, _B, 2 * EMB), lambda i: (0, i, 0)),
            pl.BlockSpec((_B, 2 * EMB), lambda i: (i, 0)),
            pl.BlockSpec((_B, 1), lambda i: (i, 0)),
            pl.BlockSpec((1, EMB), lambda i: (0, 0)),
            pl.BlockSpec((2 * EMB, HID), lambda i: (0, 0)),
            pl.BlockSpec((1, HID), lambda i: (0, 0)),
        ],
        out_specs=[
            pl.BlockSpec((_B, HID), lambda i: (i, 0)),
            pl.BlockSpec((_B, HID), lambda i: (i, 0)),
        ],
        out_shape=[
            jax.ShapeDtypeStruct((N, HID), jnp.float32),
            jax.ShapeDtypeStruct((N, HID), jnp.float32),
        ],
    )(acc, u1, dinv, b2, wm1, bm1)


@jax.jit
def kernel(x, edge_index, W1, b1, W2, b2, Wm1, bm1, Wm2, bm2):
    src = edge_index[0]
    dst = edge_index[1]
    zcol = jnp.zeros((N_PAD,), jnp.float32)
    zhid = jnp.zeros((RPS, HID), jnp.float32)

    hist = _deg_call(dst, zcol).reshape(NW, N_PAD, 1)
    u0, dinv = _tca_call(hist, x, W1)
    acc0 = _conv_call_hid(u0, src, dst, zhid)
    u1 = _tcb_call(acc0, u0, dinv, b1.reshape(1, HID), W2)
    acc1 = _conv_call_hid(u1, src, dst, zhid)
    zs, zd = _tcc_call(acc1, u1, dinv, b2.reshape(1, EMB), Wm1,
                       bm1.reshape(1, HID))
    w2 = Wm2[:, 0]
    bm2v = jnp.full((L,), bm2[0], jnp.float32)
    return _mlp_call(zs, zd, src, dst, w2, bm2v)


# speedup vs baseline: 4.9800x; 1.2051x over previous
import functools

import jax
import jax.numpy as jnp
from jax import lax
from jax.experimental import pallas as pl
from jax.experimental.pallas import tpu as pltpu
from jax.experimental.pallas import tpu_sc as plsc

N = 10000
E = 320000
IN = 128
HID = 128
EMB = 64

NC = 2
NS = 16
L = 16
NW = NC * NS
EPW = E // NW
N_PAD = 10240
RPS = N_PAD // NS
CE = 80

_mesh = plsc.VectorSubcoreMesh(core_axis_name="c", subcore_axis_name="s")


CD = 2000


def _deg_body(dst_hbm, zero_hbm, out_hbm, dchunk, hist_v):
    c = lax.axis_index("c")
    s = lax.axis_index("s")
    wid = s * NC + c
    pltpu.sync_copy(zero_hbm, hist_v)
    ones = jnp.ones((L,), jnp.float32)
    base = wid * EPW

    def chunk(ci, carry):
        pltpu.sync_copy(dst_hbm.at[pl.ds(base + ci * CD, CD)], dchunk)

        def grp(g, carry2):
            idx = dchunk[pl.ds(g * L, L)]
            plsc.addupdate_scatter(hist_v, [idx], ones)
            return carry2

        return lax.fori_loop(0, CD // L, grp, carry)

    lax.fori_loop(0, EPW // CD, chunk, 0)
    pltpu.sync_copy(hist_v, out_hbm.at[pl.ds(wid * N_PAD, N_PAD)])


_deg_call = pl.kernel(
    _deg_body,
    out_type=jax.ShapeDtypeStruct((NW * N_PAD,), jnp.float32),
    mesh=_mesh,
    compiler_params=pltpu.CompilerParams(needs_layout_passes=False),
    scratch_types=[
        pltpu.VMEM((CD,), jnp.int32),
        pltpu.VMEM((N_PAD,), jnp.float32),
    ],
)


NCH = EPW // CE
NPAIR = NCH // 2


def _conv_body(u_hbm, src_hbm, dst_hbm, zero_hbm, out_hbm,
               srcall, dstv0, dstv1, rows0, rows1,
               sg0, sg1, si0, si1, acc):
    c = lax.axis_index("c")
    s = lax.axis_index("s")
    wid = s * NC + c
    ebase = wid * EPW
    pltpu.sync_copy(zero_hbm, acc.at[pl.ds(s * RPS, RPS)])
    pltpu.sync_copy(src_hbm.at[pl.ds(ebase, EPW)], srcall)
    plsc.subcore_barrier()

    def gsrc(ci):
        return u_hbm.at[srcall.at[pl.ds(ci * CE, CE)]]

    def fill(ci, dstv, rows, si, sg):
        pltpu.async_copy(dst_hbm.at[pl.ds(ebase + ci * CE, CE)], dstv, si)
        pltpu.async_copy(gsrc(ci), rows, sg)

    def drain(dstv, rows, si, sg):
        pltpu.make_async_copy(dst_hbm.at[pl.ds(ebase, CE)], dstv, si).wait()
        pltpu.make_async_copy(gsrc(0), rows, sg).wait()

    fill(0, dstv0, rows0, si0, sg0)
    fill(1, dstv1, rows1, si1, sg1)

    def pair(ci, carry):
        a = 2 * ci
        drain(dstv0, rows0, si0, sg0)
        pltpu.sync_copy(rows0, acc.at[dstv0], add=True)
        fill(a + 2, dstv0, rows0, si0, sg0)
        drain(dstv1, rows1, si1, sg1)
        pltpu.sync_copy(rows1, acc.at[dstv1], add=True)

        @pl.when(ci < NPAIR - 1)
        def _():
            fill(a + 3, dstv1, rows1, si1, sg1)

        return carry

    lax.fori_loop(0, NPAIR, pair, 0)
    drain(dstv0, rows0, si0, sg0)
    pltpu.sync_copy(rows0, acc.at[dstv0], add=True)
    plsc.subcore_barrier()
    pltpu.sync_copy(acc.at[pl.ds(s * RPS, RPS)],
                    out_hbm.at[c, pl.ds(s * RPS, RPS)])


def _make_conv_call(d):
    return pl.kernel(
        _conv_body,
        out_type=jax.ShapeDtypeStruct((NC, N_PAD, d), jnp.float32),
        mesh=_mesh,
        scratch_types=[
            pltpu.VMEM((EPW,), jnp.int32),
            pltpu.VMEM((CE,), jnp.int32),
            pltpu.VMEM((CE,), jnp.int32),
            pltpu.VMEM((CE, d), jnp.float32),
            pltpu.VMEM((CE, d), jnp.float32),
            pltpu.SemaphoreType.DMA,
            pltpu.SemaphoreType.DMA,
            pltpu.SemaphoreType.DMA,
            pltpu.SemaphoreType.DMA,
            pltpu.MemorySpace.VMEM_SHARED((N_PAD, d), jnp.float32),
        ],
    )


_conv_call_hid = _make_conv_call(HID)


NG = CE // L
KB = 16


def _mlp_body(zs_hbm, zd_hbm, src_hbm, dst_hbm, w2_hbm, bm2_hbm, out_hbm,
              srcall, dstall, a0, b0, a1, b1, w2v, bm2v, outv,
              sg0, sg1, so):
    c = lax.axis_index("c")
    s = lax.axis_index("s")
    wid = s * NC + c
    ebase = wid * EPW
    pltpu.sync_copy(w2_hbm, w2v)
    pltpu.sync_copy(bm2_hbm, bm2v)
    pltpu.sync_copy(src_hbm.at[pl.ds(ebase, EPW)], srcall)
    pltpu.sync_copy(dst_hbm.at[pl.ds(ebase, EPW)], dstall)

    def fill(ci, ab, bb, sg):
        pltpu.async_copy(zs_hbm.at[srcall.at[pl.ds(ci * CE, CE)]], ab, sg)
        pltpu.async_copy(zd_hbm.at[dstall.at[pl.ds(ci * CE, CE)]], bb, sg)

    def drain(ab, bb, sg):
        pltpu.make_async_copy(zs_hbm.at[srcall.at[pl.ds(0, CE)]], ab, sg).wait()
        pltpu.make_async_copy(zd_hbm.at[dstall.at[pl.ds(0, CE)]], bb, sg).wait()

    iota = lax.iota(jnp.int32, L)

    def compute(ab, bb, off):
        def kblock(j, accs):
            accs = list(accs)
            for t in range(KB):
                k = j * KB + t
                kb = jnp.full((L,), k, jnp.int32)
                w2k = plsc.load_gather(w2v, [kb])
                for g in range(NG):
                    eids = iota + g * L
                    av = plsc.load_gather(ab, [eids, kb])
                    bv = plsc.load_gather(bb, [eids, kb])
                    accs[g] = accs[g] + jnp.maximum(av + bv, 0.0) * w2k
            return tuple(accs)

        accs = lax.fori_loop(
            0, HID // KB, kblock,
            tuple(jnp.zeros((L,), jnp.float32) for _ in range(NG)))
        bm2 = bm2v[...]
        for g in range(NG):
            outv[pl.ds(off + g * L, L)] = accs[g] + bm2

    fill(0, a0, b0, sg0)
    fill(1, a1, b1, sg1)

    def pair(ci, carry):
        a = 2 * ci

        @pl.when(ci > 0)
        def _():
            pltpu.make_async_copy(outv, out_hbm.at[pl.ds(ebase, 2 * CE)],
                                  so).wait()

        drain(a0, b0, sg0)
        compute(a0, b0, 0)
        fill(a + 2, a0, b0, sg0)
        drain(a1, b1, sg1)
        compute(a1, b1, CE)

        @pl.when(ci < NPAIR - 1)
        def _():
            fill(a + 3, a1, b1, sg1)

        pltpu.async_copy(outv, out_hbm.at[pl.ds(ebase + a * CE, 2 * CE)], so)
        return carry

    lax.fori_loop(0, NPAIR, pair, 0)
    pltpu.make_async_copy(outv, out_hbm.at[pl.ds(ebase, 2 * CE)], so).wait()
    drain(a0, b0, sg0)
    compute(a0, b0, 0)
    pltpu.sync_copy(outv.at[pl.ds(0, CE)],
                    out_hbm.at[pl.ds(ebase + (NCH - 1) * CE, CE)])


_mlp_call = pl.kernel(
    _mlp_body,
    out_type=jax.ShapeDtypeStruct((E,), jnp.float32),
    mesh=_mesh,
    compiler_params=pltpu.CompilerParams(needs_layout_passes=False),
    scratch_types=[
        pltpu.VMEM((EPW,), jnp.int32),
        pltpu.VMEM((EPW,), jnp.int32),
        pltpu.VMEM((CE, HID), jnp.float32),
        pltpu.VMEM((CE, HID), jnp.float32),
        pltpu.VMEM((CE, HID), jnp.float32),
        pltpu.VMEM((CE, HID), jnp.float32),
        pltpu.VMEM((HID,), jnp.float32),
        pltpu.VMEM((L,), jnp.float32),
        pltpu.VMEM((2 * CE,), jnp.float32),
        pltpu.SemaphoreType.DMA,
        pltpu.SemaphoreType.DMA,
        pltpu.SemaphoreType.DMA,
    ],
)


_B = 1000


def _tca_body(hist_ref, x_ref, w1_ref, u0_ref, dinv_ref):
    deg = jnp.sum(hist_ref[...], axis=0) + 1.0
    dinv = lax.rsqrt(deg)
    h0 = jnp.dot(x_ref[...], w1_ref[...], preferred_element_type=jnp.float32)
    u0_ref[...] = h0 * dinv
    dinv_ref[...] = dinv


def _tca_call(hist, x, w1):
    return pl.pallas_call(
        _tca_body,
        grid=(N // _B,),
        in_specs=[
            pl.BlockSpec((NW, _B, 1), lambda i: (0, i, 0)),
            pl.BlockSpec((_B, IN), lambda i: (i, 0)),
            pl.BlockSpec((IN, HID), lambda i: (0, 0)),
        ],
        out_specs=[
            pl.BlockSpec((_B, HID), lambda i: (i, 0)),
            pl.BlockSpec((_B, 1), lambda i: (i, 0)),
        ],
        out_shape=[
            jax.ShapeDtypeStruct((N, HID), jnp.float32),
            jax.ShapeDtypeStruct((N, 1), jnp.float32),
        ],
    )(hist, x, w1)


def _tcb_body(acc_ref, u0_ref, dinv_ref, b1_ref, w2_ref, u1_ref):
    dinv = dinv_ref[...]
    pre = (acc_ref[0] + acc_ref[1] + u0_ref[...]) * dinv + b1_ref[...]
    h = jnp.maximum(pre, 0.0)
    u1 = jnp.dot(h, w2_ref[...], preferred_element_type=jnp.float32) * dinv
    u1_ref[...] = jnp.concatenate([u1, jnp.zeros_like(u1)], axis=1)


def _tcb_call(acc, u0, dinv, b1, w2):
    return pl.pallas_call(
        _tcb_body,
        grid=(N // _B,),
        in_specs=[
            pl.BlockSpec((NC, _B, HID), lambda i: (0, i, 0)),
            pl.BlockSpec((_B, HID), lambda i: (i, 0)),
            pl.BlockSpec((_B, 1), lambda i: (i, 0)),
            pl.BlockSpec((1, HID), lambda i: (0, 0)),
            pl.BlockSpec((HID, EMB), lambda i: (0, 0)),
        ],
        out_specs=pl.BlockSpec((_B, 2 * EMB), lambda i: (i, 0)),
        out_shape=jax.ShapeDtypeStruct((N, 2 * EMB), jnp.float32),
    )(acc, u0, dinv, b1, w2)


def _tcc_body(acc_ref, u1_ref, dinv_ref, b2_ref, wm1_ref, bm1_ref,
              zs_ref, zd_ref):
    zw = acc_ref[0] + acc_ref[1] + u1_ref[...]
    z = zw[:, :EMB] * dinv_ref[...] + b2_ref[...]
    wm1 = wm1_ref[...]
    zs_ref[...] = jnp.dot(z, wm1[:EMB], preferred_element_type=jnp.float32)
    zd_ref[...] = (jnp.dot(z, wm1[EMB:], preferred_element_type=jnp.float32)
                   + bm1_ref[...])


def _tcc_call(acc, u1, dinv, b2, wm1, bm1):
    return pl.pallas_call(
        _tcc_body,
        grid=(N // _B,),
        in_specs=[
            pl.BlockSpec((NC, _B, 2 * EMB), lambda i: (0, i, 0)),
            pl.BlockSpec((_B, 2 * EMB), lambda i: (i, 0)),
            pl.BlockSpec((_B, 1), lambda i: (i, 0)),
            pl.BlockSpec((1, EMB), lambda i: (0, 0)),
            pl.BlockSpec((2 * EMB, HID), lambda i: (0, 0)),
            pl.BlockSpec((1, HID), lambda i: (0, 0)),
        ],
        out_specs=[
            pl.BlockSpec((_B, HID), lambda i: (i, 0)),
            pl.BlockSpec((_B, HID), lambda i: (i, 0)),
        ],
        out_shape=[
            jax.ShapeDtypeStruct((N, HID), jnp.float32),
            jax.ShapeDtypeStruct((N, HID), jnp.float32),
        ],
    )(acc, u1, dinv, b2, wm1, bm1)


@jax.jit
def kernel(x, edge_index, W1, b1, W2, b2, Wm1, bm1, Wm2, bm2):
    src = edge_index[0]
    dst = edge_index[1]
    zcol = jnp.zeros((N_PAD,), jnp.float32)
    zhid = jnp.zeros((RPS, HID), jnp.float32)

    hist = _deg_call(dst, zcol).reshape(NW, N_PAD, 1)
    u0, dinv = _tca_call(hist, x, W1)
    acc0 = _conv_call_hid(u0, src, dst, zhid)
    u1 = _tcb_call(acc0, u0, dinv, b1.reshape(1, HID), W2)
    acc1 = _conv_call_hid(u1, src, dst, zhid)
    zs, zd = _tcc_call(acc1, u1, dinv, b2.reshape(1, EMB), Wm1,
                       bm1.reshape(1, HID))
    w2 = Wm2[:, 0]
    bm2v = jnp.full((L,), bm2[0], jnp.float32)
    return _mlp_call(zs, zd, src, dst, w2, bm2v)


# speedup vs baseline: 14.1813x; 2.8477x over previous
import functools

import jax
import jax.numpy as jnp
from jax import lax
from jax.experimental import pallas as pl
from jax.experimental.pallas import tpu as pltpu
from jax.experimental.pallas import tpu_sc as plsc

N = 10000
E = 320000
IN = 128
HID = 128
EMB = 64

NC = 2
NS = 16
L = 16
NW = NC * NS
EPW = E // NW
N_PAD = 10240
RPS = N_PAD // NS
CE = 80

_mesh = plsc.VectorSubcoreMesh(core_axis_name="c", subcore_axis_name="s")


CD = 2000


def _deg_body(dst_hbm, zero_hbm, out_hbm, dchunk, hist_v):
    c = lax.axis_index("c")
    s = lax.axis_index("s")
    wid = s * NC + c
    pltpu.sync_copy(zero_hbm, hist_v)
    ones = jnp.ones((L,), jnp.float32)
    base = wid * EPW

    def chunk(ci, carry):
        pltpu.sync_copy(dst_hbm.at[pl.ds(base + ci * CD, CD)], dchunk)

        def grp(g, carry2):
            idx = dchunk[pl.ds(g * L, L)]
            plsc.addupdate_scatter(hist_v, [idx], ones)
            return carry2

        return lax.fori_loop(0, CD // L, grp, carry)

    lax.fori_loop(0, EPW // CD, chunk, 0)
    pltpu.sync_copy(hist_v, out_hbm.at[pl.ds(wid * N_PAD, N_PAD)])


_deg_call = pl.kernel(
    _deg_body,
    out_type=jax.ShapeDtypeStruct((NW * N_PAD,), jnp.float32),
    mesh=_mesh,
    compiler_params=pltpu.CompilerParams(needs_layout_passes=False),
    scratch_types=[
        pltpu.VMEM((CD,), jnp.int32),
        pltpu.VMEM((N_PAD,), jnp.float32),
    ],
)


NCH = EPW // CE
NPAIR = NCH // 2


def _conv_body(u_hbm, src_hbm, dst_hbm, zero_hbm, out_hbm,
               srcall, dstv0, dstv1, rows0, rows1,
               sg0, sg1, si0, si1, acc):
    c = lax.axis_index("c")
    s = lax.axis_index("s")
    wid = s * NC + c
    ebase = wid * EPW
    pltpu.sync_copy(zero_hbm, acc.at[pl.ds(s * RPS, RPS)])
    pltpu.sync_copy(src_hbm.at[pl.ds(ebase, EPW)], srcall)
    plsc.subcore_barrier()

    def gsrc(ci):
        return u_hbm.at[srcall.at[pl.ds(ci * CE, CE)]]

    def fill(ci, dstv, rows, si, sg):
        pltpu.async_copy(dst_hbm.at[pl.ds(ebase + ci * CE, CE)], dstv, si)
        pltpu.async_copy(gsrc(ci), rows, sg)

    def drain(dstv, rows, si, sg):
        pltpu.make_async_copy(dst_hbm.at[pl.ds(ebase, CE)], dstv, si).wait()
        pltpu.make_async_copy(gsrc(0), rows, sg).wait()

    fill(0, dstv0, rows0, si0, sg0)
    fill(1, dstv1, rows1, si1, sg1)

    def pair(ci, carry):
        a = 2 * ci
        drain(dstv0, rows0, si0, sg0)
        pltpu.sync_copy(rows0, acc.at[dstv0], add=True)
        fill(a + 2, dstv0, rows0, si0, sg0)
        drain(dstv1, rows1, si1, sg1)
        pltpu.sync_copy(rows1, acc.at[dstv1], add=True)

        @pl.when(ci < NPAIR - 1)
        def _():
            fill(a + 3, dstv1, rows1, si1, sg1)

        return carry

    lax.fori_loop(0, NPAIR, pair, 0)
    drain(dstv0, rows0, si0, sg0)
    pltpu.sync_copy(rows0, acc.at[dstv0], add=True)
    plsc.subcore_barrier()
    pltpu.sync_copy(acc.at[pl.ds(s * RPS, RPS)],
                    out_hbm.at[c, pl.ds(s * RPS, RPS)])


def _make_conv_call(d):
    return pl.kernel(
        _conv_body,
        out_type=jax.ShapeDtypeStruct((NC, N_PAD, d), jnp.float32),
        mesh=_mesh,
        scratch_types=[
            pltpu.VMEM((EPW,), jnp.int32),
            pltpu.VMEM((CE,), jnp.int32),
            pltpu.VMEM((CE,), jnp.int32),
            pltpu.VMEM((CE, d), jnp.float32),
            pltpu.VMEM((CE, d), jnp.float32),
            pltpu.SemaphoreType.DMA,
            pltpu.SemaphoreType.DMA,
            pltpu.SemaphoreType.DMA,
            pltpu.SemaphoreType.DMA,
            pltpu.MemorySpace.VMEM_SHARED((N_PAD, d), jnp.float32),
        ],
    )


_conv_call_hid = _make_conv_call(HID)


NG = CE // L
KB = 16


def _mlp_body(zs_hbm, zd_hbm, src_hbm, dst_hbm, w2_hbm, bm2_hbm, out_hbm,
              srcall, dstall, a0, b0, a1, b1, w2v, bm2v, outv,
              sg0, sg1, so):
    c = lax.axis_index("c")
    s = lax.axis_index("s")
    wid = s * NC + c
    ebase = wid * EPW
    pltpu.sync_copy(w2_hbm, w2v)
    pltpu.sync_copy(bm2_hbm, bm2v)
    pltpu.sync_copy(src_hbm.at[pl.ds(ebase, EPW)], srcall)
    pltpu.sync_copy(dst_hbm.at[pl.ds(ebase, EPW)], dstall)

    def fill(ci, ab, bb, sg):
        pltpu.async_copy(zs_hbm.at[srcall.at[pl.ds(ci * CE, CE)]], ab, sg)
        pltpu.async_copy(zd_hbm.at[dstall.at[pl.ds(ci * CE, CE)]], bb, sg)

    def drain(ab, bb, sg):
        pltpu.make_async_copy(zs_hbm.at[srcall.at[pl.ds(0, CE)]], ab, sg).wait()
        pltpu.make_async_copy(zd_hbm.at[dstall.at[pl.ds(0, CE)]], bb, sg).wait()

    iota = lax.iota(jnp.int32, L)
    w2s = [w2v[pl.ds(r * L, L)] for r in range(HID // L)]
    bm2 = bm2v[...]
    perms = [jnp.bitwise_xor(iota, jnp.int32(p)) for p in (1, 2, 4, 8)]

    def compute(ab, bb, off):
        def group(g, carry):
            ovec = jnp.zeros((L,), jnp.float32)
            for t in range(L):
                e = g * L + t
                acc = jnp.zeros((L,), jnp.float32)
                for r in range(HID // L):
                    av = ab[e, pl.ds(r * L, L)]
                    bv = bb[e, pl.ds(r * L, L)]
                    acc = acc + jnp.maximum(av + bv, 0.0) * w2s[r]
                for p in perms:
                    acc = acc + acc.at[p].get(mode="promise_in_bounds")
                ovec = jnp.where(iota == t, acc, ovec)
            outv[pl.ds(off + g * L, L)] = ovec + bm2
            return carry

        lax.fori_loop(0, NG, group, 0)

    fill(0, a0, b0, sg0)
    fill(1, a1, b1, sg1)

    def pair(ci, carry):
        a = 2 * ci

        @pl.when(ci > 0)
        def _():
            pltpu.make_async_copy(outv, out_hbm.at[pl.ds(ebase, 2 * CE)],
                                  so).wait()

        drain(a0, b0, sg0)
        compute(a0, b0, 0)
        fill(a + 2, a0, b0, sg0)
        drain(a1, b1, sg1)
        compute(a1, b1, CE)

        @pl.when(ci < NPAIR - 1)
        def _():
            fill(a + 3, a1, b1, sg1)

        pltpu.async_copy(outv, out_hbm.at[pl.ds(ebase + a * CE, 2 * CE)], so)
        return carry

    lax.fori_loop(0, NPAIR, pair, 0)
    pltpu.make_async_copy(outv, out_hbm.at[pl.ds(ebase, 2 * CE)], so).wait()
    drain(a0, b0, sg0)
    compute(a0, b0, 0)
    pltpu.sync_copy(outv.at[pl.ds(0, CE)],
                    out_hbm.at[pl.ds(ebase + (NCH - 1) * CE, CE)])


_mlp_call = pl.kernel(
    _mlp_body,
    out_type=jax.ShapeDtypeStruct((E,), jnp.float32),
    mesh=_mesh,
    compiler_params=pltpu.CompilerParams(needs_layout_passes=False),
    scratch_types=[
        pltpu.VMEM((EPW,), jnp.int32),
        pltpu.VMEM((EPW,), jnp.int32),
        pltpu.VMEM((CE, HID), jnp.float32),
        pltpu.VMEM((CE, HID), jnp.float32),
        pltpu.VMEM((CE, HID), jnp.float32),
        pltpu.VMEM((CE, HID), jnp.float32),
        pltpu.VMEM((HID,), jnp.float32),
        pltpu.VMEM((L,), jnp.float32),
        pltpu.VMEM((2 * CE,), jnp.float32),
        pltpu.SemaphoreType.DMA,
        pltpu.SemaphoreType.DMA,
        pltpu.SemaphoreType.DMA,
    ],
)


_B = 1000


def _tca_body(hist_ref, x_ref, w1_ref, u0_ref, dinv_ref):
    deg = jnp.sum(hist_ref[...], axis=0) + 1.0
    dinv = lax.rsqrt(deg)
    h0 = jnp.dot(x_ref[...], w1_ref[...], preferred_element_type=jnp.float32)
    u0_ref[...] = h0 * dinv
    dinv_ref[...] = dinv


def _tca_call(hist, x, w1):
    return pl.pallas_call(
        _tca_body,
        grid=(N // _B,),
        in_specs=[
            pl.BlockSpec((NW, _B, 1), lambda i: (0, i, 0)),
            pl.BlockSpec((_B, IN), lambda i: (i, 0)),
            pl.BlockSpec((IN, HID), lambda i: (0, 0)),
        ],
        out_specs=[
            pl.BlockSpec((_B, HID), lambda i: (i, 0)),
            pl.BlockSpec((_B, 1), lambda i: (i, 0)),
        ],
        out_shape=[
            jax.ShapeDtypeStruct((N, HID), jnp.float32),
            jax.ShapeDtypeStruct((N, 1), jnp.float32),
        ],
    )(hist, x, w1)


def _tcb_body(acc_ref, u0_ref, dinv_ref, b1_ref, w2_ref, u1_ref):
    dinv = dinv_ref[...]
    pre = (acc_ref[0] + acc_ref[1] + u0_ref[...]) * dinv + b1_ref[...]
    h = jnp.maximum(pre, 0.0)
    u1 = jnp.dot(h, w2_ref[...], preferred_element_type=jnp.float32) * dinv
    u1_ref[...] = jnp.concatenate([u1, jnp.zeros_like(u1)], axis=1)


def _tcb_call(acc, u0, dinv, b1, w2):
    return pl.pallas_call(
        _tcb_body,
        grid=(N // _B,),
        in_specs=[
            pl.BlockSpec((NC, _B, HID), lambda i: (0, i, 0)),
            pl.BlockSpec((_B, HID), lambda i: (i, 0)),
            pl.BlockSpec((_B, 1), lambda i: (i, 0)),
            pl.BlockSpec((1, HID), lambda i: (0, 0)),
            pl.BlockSpec((HID, EMB), lambda i: (0, 0)),
        ],
        out_specs=pl.BlockSpec((_B, 2 * EMB), lambda i: (i, 0)),
        out_shape=jax.ShapeDtypeStruct((N, 2 * EMB), jnp.float32),
    )(acc, u0, dinv, b1, w2)


def _tcc_body(acc_ref, u1_ref, dinv_ref, b2_ref, wm1_ref, bm1_ref,
              zs_ref, zd_ref):
    zw = acc_ref[0] + acc_ref[1] + u1_ref[...]
    z = zw[:, :EMB] * dinv_ref[...] + b2_ref[...]
    wm1 = wm1_ref[...]
    zs_ref[...] = jnp.dot(z, wm1[:EMB], preferred_element_type=jnp.float32)
    zd_ref[...] = (jnp.dot(z, wm1[EMB:], preferred_element_type=jnp.float32)
                   + bm1_ref[...])


def _tcc_call(acc, u1, dinv, b2, wm1, bm1):
    return pl.pallas_call(
        _tcc_body,
        grid=(N // _B,),
        in_specs=[
            pl.BlockSpec((NC, _B, 2 * EMB), lambda i: (0, i, 0)),
            pl.BlockSpec((_B, 2 * EMB), lambda i: (i, 0)),
            pl.BlockSpec((_B, 1), lambda i: (i, 0)),
            pl.BlockSpec((1, EMB), lambda i: (0, 0)),
            pl.BlockSpec((2 * EMB, HID), lambda i: (0, 0)),
            pl.BlockSpec((1, HID), lambda i: (0, 0)),
        ],
        out_specs=[
            pl.BlockSpec((_B, HID), lambda i: (i, 0)),
            pl.BlockSpec((_B, HID), lambda i: (i, 0)),
        ],
        out_shape=[
            jax.ShapeDtypeStruct((N, HID), jnp.float32),
            jax.ShapeDtypeStruct((N, HID), jnp.float32),
        ],
    )(acc, u1, dinv, b2, wm1, bm1)


@jax.jit
def kernel(x, edge_index, W1, b1, W2, b2, Wm1, bm1, Wm2, bm2):
    src = edge_index[0]
    dst = edge_index[1]
    zcol = jnp.zeros((N_PAD,), jnp.float32)
    zhid = jnp.zeros((RPS, HID), jnp.float32)

    hist = _deg_call(dst, zcol).reshape(NW, N_PAD, 1)
    u0, dinv = _tca_call(hist, x, W1)
    acc0 = _conv_call_hid(u0, src, dst, zhid)
    u1 = _tcb_call(acc0, u0, dinv, b1.reshape(1, HID), W2)
    acc1 = _conv_call_hid(u1, src, dst, zhid)
    zs, zd = _tcc_call(acc1, u1, dinv, b2.reshape(1, EMB), Wm1,
                       bm1.reshape(1, HID))
    w2 = Wm2[:, 0]
    bm2v = jnp.full((L,), bm2[0], jnp.float32)
    return _mlp_call(zs, zd, src, dst, w2, bm2v)
